# baseline (speedup 1.0000x reference)
import jax
import jax.numpy as jnp
from jax import lax
from jax.experimental import pallas as pl
from jax.experimental.pallas import tpu as pltpu
from jax.experimental.pallas import tpu_sc as plsc

N = 10000
E = 320000
D_BIG = 128
D_SMALL = 16
NC = 2
NS = 16
CHUNK = 128
WIN = 8
NCHUNK = E // CHUNK
ROWS_PER_TILE = N // NS

ICH = 64
R_PAD = 10240
ICPW = R_PAD // (NC * NS * ICH)


def _sc_body(ops_hbm, items_hbm, mats_hbm, ress_hbm,
             g_pred, s_pred, g_succ, s_succ,
             g_mat, s_mat, g_res, s_res,
             ri_hbm, zeros_big, zeros_small,
             out_pred, out_succ, out_mat, out_res, out_items,
             gidx, sidx, rows_a, rows_b, rows_sa, rows_sb, iidx,
             acc_big, acc_small, sem_a, sem_b):
  c = lax.axis_index("c")
  s = lax.axis_index("s")

  zb = s * ROWS_PER_TILE
  pltpu.sync_copy(zeros_big.at[pl.ds(zb, ROWS_PER_TILE)],
                  acc_big.at[pl.ds(zb, ROWS_PER_TILE)])
  pltpu.sync_copy(zeros_small.at[pl.ds(zb, ROWS_PER_TILE)],
                  acc_small.at[pl.ds(zb, ROWS_PER_TILE)])
  plsc.subcore_barrier()

  lo = (s * NCHUNK) // NS
  hi = ((s + 1) * NCHUNK) // NS
  n_chunks = hi - lo
  n_win = n_chunks // WIN

  def seg_pass(g_hbm, s_hbm, table_hbm, acc, ra, rb):
    def window(wi, _):
      base = lo + wi * WIN
      pltpu.sync_copy(g_hbm.at[pl.ds(base, WIN)], gidx)
      pltpu.sync_copy(s_hbm.at[pl.ds(base, WIN)], sidx)
      loff = ((lo + wi * WIN + s) % 78) * CHUNK
      pltpu.async_copy(table_hbm.at[pl.ds(loff, CHUNK)], ra, sem_a)

      def pair(j2, _):
        e = 2 * j2
        pltpu.async_copy(table_hbm.at[pl.ds(loff, CHUNK)], rb, sem_b)
        pltpu.make_async_copy(table_hbm.at[pl.ds(loff, CHUNK)], ra, sem_a).wait()
        pltpu.sync_copy(ra, acc.at[sidx.at[e]], add=True)

        @pl.when(e + 2 < WIN)
        def _():
          pltpu.async_copy(table_hbm.at[pl.ds(loff, CHUNK)], ra, sem_a)
        pltpu.make_async_copy(table_hbm.at[pl.ds(loff, CHUNK)], rb, sem_b).wait()
        pltpu.sync_copy(rb, acc.at[sidx.at[e + 1]], add=True)
        return ()
      lax.fori_loop(0, WIN // 2, pair, (), unroll=False)
      return ()
    lax.fori_loop(0, n_win, window, (), unroll=False)

    def tail(t, _):
      ci = lo + n_win * WIN + t
      pltpu.sync_copy(g_hbm.at[pl.ds(ci, 1)], gidx.at[pl.ds(0, 1)])
      pltpu.sync_copy(s_hbm.at[pl.ds(ci, 1)], sidx.at[pl.ds(0, 1)])
      pltpu.async_copy(table_hbm.at[gidx.at[0]], ra, sem_a).wait()
      pltpu.sync_copy(ra, acc.at[sidx.at[0]], add=True)
      return ()
    lax.fori_loop(0, n_chunks - n_win * WIN, tail, (), unroll=False)

  @pl.when(c == 0)
  def _():
    seg_pass(g_pred, s_pred, ops_hbm, acc_big, rows_a, rows_b)
    seg_pass(g_mat, s_mat, mats_hbm, acc_small, rows_sa, rows_sb)

  @pl.when(c == 1)
  def _():
    seg_pass(g_succ, s_succ, ops_hbm, acc_big, rows_a, rows_b)
    seg_pass(g_res, s_res, ress_hbm, acc_small, rows_sa, rows_sb)

  w = c * NS + s
  irows = rows_a.at[pl.ds(0, ICH)]
  pltpu.sync_copy(ri_hbm.at[w], iidx)

  def ibody(k, _):
    pltpu.async_copy(items_hbm.at[iidx.at[k]], irows, sem_a).wait()
    pltpu.sync_copy(irows, out_items.at[pl.ds((w * ICPW + k) * ICH, ICH)])
    return ()
  lax.fori_loop(0, ICPW, ibody, (), unroll=False)

  plsc.subcore_barrier()

  @pl.when(c == 0)
  def _():
    pltpu.sync_copy(acc_big.at[pl.ds(zb, ROWS_PER_TILE)],
                    out_pred.at[pl.ds(zb, ROWS_PER_TILE)])
    pltpu.sync_copy(acc_small.at[pl.ds(zb, ROWS_PER_TILE)],
                    out_mat.at[pl.ds(zb, ROWS_PER_TILE)])

  @pl.when(c == 1)
  def _():
    pltpu.sync_copy(acc_big.at[pl.ds(zb, ROWS_PER_TILE)],
                    out_succ.at[pl.ds(zb, ROWS_PER_TILE)])
    pltpu.sync_copy(acc_small.at[pl.ds(zb, ROWS_PER_TILE)],
                    out_res.at[pl.ds(zb, ROWS_PER_TILE)])


_sc_call = pl.kernel(
    _sc_body,
    out_type=(
        jax.ShapeDtypeStruct((N, D_BIG), jnp.float32),
        jax.ShapeDtypeStruct((N, D_BIG), jnp.float32),
        jax.ShapeDtypeStruct((N, D_SMALL), jnp.float32),
        jax.ShapeDtypeStruct((N, D_SMALL), jnp.float32),
        jax.ShapeDtypeStruct((R_PAD, D_BIG), jnp.float32),
    ),
    mesh=plsc.VectorSubcoreMesh(core_axis_name="c", subcore_axis_name="s",
                                num_cores=NC, num_subcores=NS),
    scratch_types=(
        pltpu.VMEM((WIN, CHUNK), jnp.int32),
        pltpu.VMEM((WIN, CHUNK), jnp.int32),
        pltpu.VMEM((CHUNK, D_BIG), jnp.float32),
        pltpu.VMEM((CHUNK, D_BIG), jnp.float32),
        pltpu.VMEM((CHUNK, D_SMALL), jnp.float32),
        pltpu.VMEM((CHUNK, D_SMALL), jnp.float32),
        pltpu.VMEM((ICPW, ICH), jnp.int32),
        pltpu.VMEM_SHARED((N, D_BIG), jnp.float32),
        pltpu.VMEM_SHARED((N, D_SMALL), jnp.float32),
        pltpu.SemaphoreType.DMA,
        pltpu.SemaphoreType.DMA,
    ),
    compiler_params=pltpu.CompilerParams(use_tc_tiling_on_sc=False),
)


def _tc_body(ops, item_rows, agg_pred, agg_succ, agg_mat, agg_res,
             w1s, b1s, w2s, b2s,
             w1i, b1i, w2i, b2i,
             w1p, b1p, w2p, b2p,
             w1u, b1u, w2u, b2u,
             w1r, b1r, w2r, b2r,
             w1m, b1m, w2m, b2m,
             a_p, a_u, a_r, a_m, a_i, a_s, b1c, w2c, b2c, w3c, b3c,
             out):
  f32 = jnp.float32

  def mlp2(x, w1, b1, w2, b2):
    h = jnp.maximum(jnp.dot(x[...], w1[...], preferred_element_type=f32)
                    + b1[...], 0.0)
    return jnp.dot(h, w2[...], preferred_element_type=f32) + b2[...]

  pred_e = mlp2(agg_pred, w1p, b1p, w2p, b2p)
  succ_e = mlp2(agg_succ, w1u, b1u, w2u, b2u)
  res_e = mlp2(agg_res, w1r, b1r, w2r, b2r)
  mat_e = mlp2(agg_mat, w1m, b1m, w2m, b2m)
  item_e = mlp2(item_rows, w1i, b1i, w2i, b2i)
  self_e = mlp2(ops, w1s, b1s, w2s, b2s)

  h = (jnp.dot(pred_e, a_p[...], preferred_element_type=f32)
       + jnp.dot(succ_e, a_u[...], preferred_element_type=f32)
       + jnp.dot(res_e, a_r[...], preferred_element_type=f32)
       + jnp.dot(mat_e, a_m[...], preferred_element_type=f32)
       + jnp.dot(item_e, a_i[...], preferred_element_type=f32)
       + jnp.dot(self_e, a_s[...], preferred_element_type=f32)
       + b1c[...])
  h = jnp.maximum(h, 0.0)
  h = jnp.maximum(jnp.dot(h, w2c[...], preferred_element_type=f32) + b2c[...],
                  0.0)
  out[...] = jnp.dot(h, w3c[...], preferred_element_type=f32) + b3c[...]


_TC_BLOCK = 1000
_TC_GRID = N // _TC_BLOCK


def _row_spec(d):
  return pl.BlockSpec((_TC_BLOCK, d), lambda i: (i, 0))


def _full_spec(shape):
  return pl.BlockSpec(shape, lambda i: (0,) * len(shape))


def kernel(operations, items, related_items, materials, resources,
           need_for_resources, need_for_materials, precedences, params):
  g_pred = precedences[1].reshape(NCHUNK, CHUNK)
  s_pred = precedences[0].reshape(NCHUNK, CHUNK)
  g_mat = need_for_materials[1].reshape(NCHUNK, CHUNK)
  s_mat = need_for_materials[0].reshape(NCHUNK, CHUNK)
  g_res = need_for_resources[1].reshape(NCHUNK, CHUNK)
  s_res = need_for_resources[0].reshape(NCHUNK, CHUNK)
  ri = jnp.concatenate(
      [related_items,
       jnp.zeros((R_PAD - N,), jnp.int32)]).reshape(NC * NS, ICPW, ICH)
  zeros_big = jnp.zeros((N, D_BIG), jnp.float32)
  zeros_small = jnp.zeros((N, D_SMALL), jnp.float32)

  agg_pred, agg_succ, agg_mat, agg_res, item_rows = _sc_call(
      operations, items, materials, resources,
      g_pred, s_pred, s_pred, g_pred,
      g_mat, s_mat, g_res, s_res,
      ri, zeros_big, zeros_small)

  p = params
  c = p['comb']
  a_p = c['W1'][0:128]
  a_u = c['W1'][128:256]
  a_r = c['W1'][256:272]
  a_m = c['W1'][272:288]
  a_i = c['W1'][288:416]
  a_s = c['W1'][416:544]

  def b2d(b):
    return b.reshape(1, -1)

  mlp_args = []
  for name in ('self', 'items', 'pred', 'succ', 'res', 'mat'):
    q = p[name]
    mlp_args += [q['W1'], b2d(q['b1']), q['W2'], b2d(q['b2'])]

  comb_args = [a_p, a_u, a_r, a_m, a_i, a_s, b2d(c['b1']),
               c['W2'], b2d(c['b2']), c['W3'], b2d(c['b3'])]

  din_specs = [_row_spec(D_BIG), _row_spec(D_BIG), _row_spec(D_BIG),
               _row_spec(D_BIG), _row_spec(D_SMALL), _row_spec(D_SMALL)]
  w_specs = []
  for a in mlp_args + comb_args:
    w_specs.append(_full_spec(a.shape))

  out = pl.pallas_call(
      _tc_body,
      grid=(_TC_GRID,),
      in_specs=din_specs + w_specs,
      out_specs=_row_spec(D_BIG),
      out_shape=jax.ShapeDtypeStruct((N, D_BIG), jnp.float32),
  )(operations, item_rows, agg_pred, agg_succ, agg_mat, agg_res,
    *mlp_args, *comb_args)
  return out


# speedup vs baseline: 1.0414x; 1.0414x over previous
import jax
import jax.numpy as jnp
from jax import lax
from jax.experimental import pallas as pl
from jax.experimental.pallas import tpu as pltpu
from jax.experimental.pallas import tpu_sc as plsc

N = 10000
E = 320000
D_BIG = 128
D_SMALL = 16
NC = 2
NS = 16
CHUNK = 64
NBUF = 4
WIN = 24
NCHUNK = E // CHUNK
ROWS_PER_TILE = N // NS

ICH = 64
R_PAD = 10240
ICPW = R_PAD // (NC * NS * ICH)


def _sc_body(ops_hbm, items_hbm, mats_hbm, ress_hbm,
             g_pred, s_pred, g_succ, s_succ,
             g_mat, s_mat, g_res, s_res,
             ri_hbm, zeros_big, zeros_small,
             out_pred, out_succ, out_mat, out_res, out_items,
             gidx, sidx, r0, r1, r2, r3, q0, q1, q2, q3, iidx,
             acc_big, acc_small,
             sg0, sg1, sg2, sg3, ss0, ss1, ss2, ss3):
  c = lax.axis_index("c")
  s = lax.axis_index("s")
  sg = (sg0, sg1, sg2, sg3)
  ss = (ss0, ss1, ss2, ss3)

  zb = s * ROWS_PER_TILE
  pltpu.sync_copy(zeros_big.at[pl.ds(zb, ROWS_PER_TILE)],
                  acc_big.at[pl.ds(zb, ROWS_PER_TILE)])
  pltpu.sync_copy(zeros_small.at[pl.ds(zb, ROWS_PER_TILE)],
                  acc_small.at[pl.ds(zb, ROWS_PER_TILE)])
  plsc.subcore_barrier()

  lo = (s * NCHUNK) // NS
  hi = ((s + 1) * NCHUNK) // NS
  n_chunks = hi - lo
  n_win = n_chunks // WIN

  def seg_pass(g_hbm, s_hbm, table_hbm, acc, rs):
    def window(wi, _):
      base = lo + wi * WIN
      pltpu.sync_copy(g_hbm.at[pl.ds(base, WIN)], gidx)
      pltpu.sync_copy(s_hbm.at[pl.ds(base, WIN)], sidx)
      for b in range(NBUF):
        pltpu.async_copy(table_hbm.at[gidx.at[b]], rs[b], sg[b])

      def quad(q, _):
        j0 = q * NBUF
        for b in range(NBUF):
          j = j0 + b
          pltpu.make_async_copy(table_hbm.at[gidx.at[j]], rs[b],
                                sg[b]).wait()
          pltpu.async_copy(rs[b], acc.at[sidx.at[j]], ss[b], add=True)
        for b in range(NBUF):
          jn = j0 + b + NBUF

          @pl.when(jn < WIN)
          def _():
            pltpu.make_async_copy(rs[b], acc.at[sidx.at[j0 + b]],
                                  ss[b]).wait()
            pltpu.async_copy(table_hbm.at[gidx.at[jn]], rs[b], sg[b])
        return ()
      lax.fori_loop(0, WIN // NBUF, quad, (), unroll=False)
      for b in range(NBUF):
        pltpu.make_async_copy(rs[b], acc.at[sidx.at[b]], ss[b]).wait()
      return ()
    lax.fori_loop(0, n_win, window, (), unroll=False)

    def tail(t, _):
      ci = lo + n_win * WIN + t
      pltpu.sync_copy(g_hbm.at[pl.ds(ci, 1)], gidx.at[pl.ds(0, 1)])
      pltpu.sync_copy(s_hbm.at[pl.ds(ci, 1)], sidx.at[pl.ds(0, 1)])
      pltpu.async_copy(table_hbm.at[gidx.at[0]], rs[0], sg[0]).wait()
      pltpu.sync_copy(rs[0], acc.at[sidx.at[0]], add=True)
      return ()
    lax.fori_loop(0, n_chunks - n_win * WIN, tail, (), unroll=False)

  @pl.when(c == 0)
  def _():
    seg_pass(g_pred, s_pred, ops_hbm, acc_big, (r0, r1, r2, r3))
    seg_pass(g_mat, s_mat, mats_hbm, acc_small, (q0, q1, q2, q3))

  @pl.when(c == 1)
  def _():
    seg_pass(g_succ, s_succ, ops_hbm, acc_big, (r0, r1, r2, r3))
    seg_pass(g_res, s_res, ress_hbm, acc_small, (q0, q1, q2, q3))

  w = c * NS + s
  pltpu.sync_copy(ri_hbm.at[w], iidx)

  def ibody(k, _):
    pltpu.async_copy(items_hbm.at[iidx.at[k]], r0, sg[0]).wait()
    pltpu.sync_copy(r0, out_items.at[pl.ds((w * ICPW + k) * ICH, ICH)])
    return ()
  lax.fori_loop(0, ICPW, ibody, (), unroll=False)

  plsc.subcore_barrier()

  @pl.when(c == 0)
  def _():
    pltpu.sync_copy(acc_big.at[pl.ds(zb, ROWS_PER_TILE)],
                    out_pred.at[pl.ds(zb, ROWS_PER_TILE)])
    pltpu.sync_copy(acc_small.at[pl.ds(zb, ROWS_PER_TILE)],
                    out_mat.at[pl.ds(zb, ROWS_PER_TILE)])

  @pl.when(c == 1)
  def _():
    pltpu.sync_copy(acc_big.at[pl.ds(zb, ROWS_PER_TILE)],
                    out_succ.at[pl.ds(zb, ROWS_PER_TILE)])
    pltpu.sync_copy(acc_small.at[pl.ds(zb, ROWS_PER_TILE)],
                    out_res.at[pl.ds(zb, ROWS_PER_TILE)])


_sc_call = pl.kernel(
    _sc_body,
    out_type=(
        jax.ShapeDtypeStruct((N, D_BIG), jnp.float32),
        jax.ShapeDtypeStruct((N, D_BIG), jnp.float32),
        jax.ShapeDtypeStruct((N, D_SMALL), jnp.float32),
        jax.ShapeDtypeStruct((N, D_SMALL), jnp.float32),
        jax.ShapeDtypeStruct((R_PAD, D_BIG), jnp.float32),
    ),
    mesh=plsc.VectorSubcoreMesh(core_axis_name="c", subcore_axis_name="s",
                                num_cores=NC, num_subcores=NS),
    scratch_types=(
        pltpu.VMEM((WIN, CHUNK), jnp.int32),
        pltpu.VMEM((WIN, CHUNK), jnp.int32),
        pltpu.VMEM((ICH, D_BIG), jnp.float32),
        pltpu.VMEM((CHUNK, D_BIG), jnp.float32),
        pltpu.VMEM((CHUNK, D_BIG), jnp.float32),
        pltpu.VMEM((CHUNK, D_BIG), jnp.float32),
        pltpu.VMEM((CHUNK, D_SMALL), jnp.float32),
        pltpu.VMEM((CHUNK, D_SMALL), jnp.float32),
        pltpu.VMEM((CHUNK, D_SMALL), jnp.float32),
        pltpu.VMEM((CHUNK, D_SMALL), jnp.float32),
        pltpu.VMEM((ICPW, ICH), jnp.int32),
        pltpu.VMEM_SHARED((N, D_BIG), jnp.float32),
        pltpu.VMEM_SHARED((N, D_SMALL), jnp.float32),
        pltpu.SemaphoreType.DMA,
        pltpu.SemaphoreType.DMA,
        pltpu.SemaphoreType.DMA,
        pltpu.SemaphoreType.DMA,
        pltpu.SemaphoreType.DMA,
        pltpu.SemaphoreType.DMA,
        pltpu.SemaphoreType.DMA,
        pltpu.SemaphoreType.DMA,
    ),
    compiler_params=pltpu.CompilerParams(use_tc_tiling_on_sc=False),
)


def _tc_body(ops, item_rows, agg_pred, agg_succ, agg_mat, agg_res,
             w1s, b1s, w2s, b2s,
             w1i, b1i, w2i, b2i,
             w1p, b1p, w2p, b2p,
             w1u, b1u, w2u, b2u,
             w1r, b1r, w2r, b2r,
             w1m, b1m, w2m, b2m,
             a_p, a_u, a_r, a_m, a_i, a_s, b1c, w2c, b2c, w3c, b3c,
             out):
  f32 = jnp.float32

  def mlp2(x, w1, b1, w2, b2):
    h = jnp.maximum(jnp.dot(x[...], w1[...], preferred_element_type=f32)
                    + b1[...], 0.0)
    return jnp.dot(h, w2[...], preferred_element_type=f32) + b2[...]

  pred_e = mlp2(agg_pred, w1p, b1p, w2p, b2p)
  succ_e = mlp2(agg_succ, w1u, b1u, w2u, b2u)
  res_e = mlp2(agg_res, w1r, b1r, w2r, b2r)
  mat_e = mlp2(agg_mat, w1m, b1m, w2m, b2m)
  item_e = mlp2(item_rows, w1i, b1i, w2i, b2i)
  self_e = mlp2(ops, w1s, b1s, w2s, b2s)

  h = (jnp.dot(pred_e, a_p[...], preferred_element_type=f32)
       + jnp.dot(succ_e, a_u[...], preferred_element_type=f32)
       + jnp.dot(res_e, a_r[...], preferred_element_type=f32)
       + jnp.dot(mat_e, a_m[...], preferred_element_type=f32)
       + jnp.dot(item_e, a_i[...], preferred_element_type=f32)
       + jnp.dot(self_e, a_s[...], preferred_element_type=f32)
       + b1c[...])
  h = jnp.maximum(h, 0.0)
  h = jnp.maximum(jnp.dot(h, w2c[...], preferred_element_type=f32) + b2c[...],
                  0.0)
  out[...] = jnp.dot(h, w3c[...], preferred_element_type=f32) + b3c[...]


_TC_BLOCK = 1000
_TC_GRID = N // _TC_BLOCK


def _row_spec(d):
  return pl.BlockSpec((_TC_BLOCK, d), lambda i: (i, 0))


def _full_spec(shape):
  return pl.BlockSpec(shape, lambda i: (0,) * len(shape))


def kernel(operations, items, related_items, materials, resources,
           need_for_resources, need_for_materials, precedences, params):
  g_pred = precedences[1].reshape(NCHUNK, CHUNK)
  s_pred = precedences[0].reshape(NCHUNK, CHUNK)
  g_mat = need_for_materials[1].reshape(NCHUNK, CHUNK)
  s_mat = need_for_materials[0].reshape(NCHUNK, CHUNK)
  g_res = need_for_resources[1].reshape(NCHUNK, CHUNK)
  s_res = need_for_resources[0].reshape(NCHUNK, CHUNK)
  ri = jnp.concatenate(
      [related_items,
       jnp.zeros((R_PAD - N,), jnp.int32)]).reshape(NC * NS, ICPW, ICH)
  zeros_big = jnp.zeros((N, D_BIG), jnp.float32)
  zeros_small = jnp.zeros((N, D_SMALL), jnp.float32)

  agg_pred, agg_succ, agg_mat, agg_res, item_rows = _sc_call(
      operations, items, materials, resources,
      g_pred, s_pred, s_pred, g_pred,
      g_mat, s_mat, g_res, s_res,
      ri, zeros_big, zeros_small)

  p = params
  c = p['comb']
  a_p = c['W1'][0:128]
  a_u = c['W1'][128:256]
  a_r = c['W1'][256:272]
  a_m = c['W1'][272:288]
  a_i = c['W1'][288:416]
  a_s = c['W1'][416:544]

  def b2d(b):
    return b.reshape(1, -1)

  mlp_args = []
  for name in ('self', 'items', 'pred', 'succ', 'res', 'mat'):
    q = p[name]
    mlp_args += [q['W1'], b2d(q['b1']), q['W2'], b2d(q['b2'])]

  comb_args = [a_p, a_u, a_r, a_m, a_i, a_s, b2d(c['b1']),
               c['W2'], b2d(c['b2']), c['W3'], b2d(c['b3'])]

  din_specs = [_row_spec(D_BIG), _row_spec(D_BIG), _row_spec(D_BIG),
               _row_spec(D_BIG), _row_spec(D_SMALL), _row_spec(D_SMALL)]
  w_specs = []
  for a in mlp_args + comb_args:
    w_specs.append(_full_spec(a.shape))

  out = pl.pallas_call(
      _tc_body,
      grid=(_TC_GRID,),
      in_specs=din_specs + w_specs,
      out_specs=_row_spec(D_BIG),
      out_shape=jax.ShapeDtypeStruct((N, D_BIG), jnp.float32),
  )(operations, item_rows, agg_pred, agg_succ, agg_mat, agg_res,
    *mlp_args, *comb_args)
  return out
